# Initial kernel scaffold; baseline (speedup 1.0000x reference)
import jax
import jax.numpy as jnp
from jax.experimental import pallas as pl


def kernel(xyz1, xyz2, points1, points2, W0, b0, g0, beta0, W1, b1, g1, beta1):
    raise NotImplementedError("write your pallas kernel here")


# speedup vs baseline: 13.6462x; 13.6462x over previous
import functools

import jax
import jax.numpy as jnp
from jax.experimental import pallas as pl
from jax.experimental.pallas import tpu as pltpu

B, N, M = 16, 1024, 256
C1, C2 = 256, 256
OUT0, OUT1 = 256, 256
NROWS = B * N
ROWBLK = 2048
EPS_BN = 1e-5
EPS_D = 1e-8


def _k1_body(x1_ref, x2t_ref, p1_ref, p2_ref, w0at_ref, w0bt_ref, b0_ref,
             y0_ref, stats_ref):
    b = pl.program_id(0)

    x1 = x1_ref[0]
    x2t = x2t_ref[0]
    x1sq = jnp.sum(x1 * x1, axis=1, keepdims=True)
    x2sq = jnp.sum(x2t * x2t, axis=0, keepdims=True)
    cross = jax.lax.dot_general(
        x1, x2t, (((1,), (0,)), ((), ())),
        preferred_element_type=jnp.float32,
        precision=jax.lax.Precision.DEFAULT)
    d2 = jnp.maximum(x1sq + x2sq - 2.0 * cross, 0.0)

    iota_m = jax.lax.broadcasted_iota(jnp.int32, (N, M), 1)
    big = jnp.float32(3.4e38)
    s = jnp.zeros((N, M), jnp.float32)
    recip_sum = jnp.zeros((N, 1), jnp.float32)
    recips = []
    masks = []
    for _ in range(3):
        mval = jnp.min(d2, axis=1, keepdims=True)
        sel = jnp.min(jnp.where(d2 == mval, iota_m, M), axis=1,
                      keepdims=True)
        hit = iota_m == sel
        r = 1.0 / (mval + EPS_D)
        recip_sum = recip_sum + r
        recips.append(r)
        masks.append(hit)
        d2 = jnp.where(hit, big, d2)
    inv_norm = 1.0 / recip_sum
    for r, hit in zip(recips, masks):
        s = jnp.where(hit, r * inv_norm, s)

    interp = jax.lax.dot_general(
        s, p2_ref[0], (((1,), (0,)), ((), ())),
        preferred_element_type=jnp.float32)

    y0 = (jax.lax.dot_general(p1_ref[0], w0at_ref[...],
                              (((1,), (0,)), ((), ())),
                              preferred_element_type=jnp.float32)
          + jax.lax.dot_general(interp, w0bt_ref[...],
                                (((1,), (0,)), ((), ())),
                                preferred_element_type=jnp.float32)
          + b0_ref[...])
    y0_ref[0] = y0

    @pl.when(b == 0)
    def _init():
        stats_ref[...] = jnp.zeros_like(stats_ref)

    part = jnp.concatenate(
        [jnp.sum(y0, axis=0, keepdims=True),
         jnp.sum(y0 * y0, axis=0, keepdims=True)], axis=0)
    stats_ref[...] += part


def _k2_body(y0_ref, stats0_ref, w1t_ref, b1_ref, g0_ref, beta0_ref,
             y1_ref, stats1_ref):
    i = pl.program_id(0)
    inv_n = jnp.float32(1.0 / NROWS)
    mean = stats0_ref[0:1, :] * inv_n
    var = stats0_ref[1:2, :] * inv_n - mean * mean
    scale = g0_ref[...] * jax.lax.rsqrt(var + EPS_BN)
    shift = beta0_ref[...] - mean * scale
    h = jnp.maximum(y0_ref[...] * scale + shift, 0.0)
    y1 = jax.lax.dot_general(h, w1t_ref[...], (((1,), (0,)), ((), ())),
                             preferred_element_type=jnp.float32) + b1_ref[...]
    y1_ref[...] = y1

    @pl.when(i == 0)
    def _init():
        stats1_ref[...] = jnp.zeros_like(stats1_ref)

    part = jnp.concatenate(
        [jnp.sum(y1, axis=0, keepdims=True),
         jnp.sum(y1 * y1, axis=0, keepdims=True)], axis=0)
    stats1_ref[...] += part


def _k3_body(y1_ref, stats1_ref, g1_ref, beta1_ref, out_ref):
    inv_n = jnp.float32(1.0 / NROWS)
    mean = stats1_ref[0:1, :] * inv_n
    var = stats1_ref[1:2, :] * inv_n - mean * mean
    scale = g1_ref[...] * jax.lax.rsqrt(var + EPS_BN)
    shift = beta1_ref[...] - mean * scale
    out_ref[...] = jnp.maximum(y1_ref[...] * scale + shift, 0.0)


@jax.jit
def kernel(xyz1, xyz2, points1, points2, W0, b0, g0, beta0, W1, b1, g1, beta1):
    f32 = jnp.float32
    x1p = jnp.pad(xyz1, ((0, 0), (0, 0), (0, 5)))
    x2t = jnp.pad(xyz2, ((0, 0), (0, 0), (0, 5))).transpose(0, 2, 1)
    w0t = W0.T
    w0at, w0bt = w0t[:C1], w0t[C1:]
    w1t = W1.T
    row = lambda v: v.reshape(1, -1)

    y0, stats0 = pl.pallas_call(
        _k1_body,
        grid=(B,),
        in_specs=[
            pl.BlockSpec((1, N, 8), lambda b: (b, 0, 0)),
            pl.BlockSpec((1, 8, M), lambda b: (b, 0, 0)),
            pl.BlockSpec((1, N, C1), lambda b: (b, 0, 0)),
            pl.BlockSpec((1, M, C2), lambda b: (b, 0, 0)),
            pl.BlockSpec((C1, OUT0), lambda b: (0, 0)),
            pl.BlockSpec((C2, OUT0), lambda b: (0, 0)),
            pl.BlockSpec((1, OUT0), lambda b: (0, 0)),
        ],
        out_specs=[
            pl.BlockSpec((1, N, OUT0), lambda b: (b, 0, 0)),
            pl.BlockSpec((2, OUT0), lambda b: (0, 0)),
        ],
        out_shape=[
            jax.ShapeDtypeStruct((B, N, OUT0), f32),
            jax.ShapeDtypeStruct((2, OUT0), f32),
        ],
    )(x1p, x2t, points1, points2, w0at, w0bt, row(b0))

    y0f = y0.reshape(NROWS, OUT0)
    nblk = NROWS // ROWBLK
    y1, stats1 = pl.pallas_call(
        _k2_body,
        grid=(nblk,),
        in_specs=[
            pl.BlockSpec((ROWBLK, OUT0), lambda i: (i, 0)),
            pl.BlockSpec((2, OUT0), lambda i: (0, 0)),
            pl.BlockSpec((OUT0, OUT1), lambda i: (0, 0)),
            pl.BlockSpec((1, OUT1), lambda i: (0, 0)),
            pl.BlockSpec((1, OUT0), lambda i: (0, 0)),
            pl.BlockSpec((1, OUT0), lambda i: (0, 0)),
        ],
        out_specs=[
            pl.BlockSpec((ROWBLK, OUT1), lambda i: (i, 0)),
            pl.BlockSpec((2, OUT1), lambda i: (0, 0)),
        ],
        out_shape=[
            jax.ShapeDtypeStruct((NROWS, OUT1), f32),
            jax.ShapeDtypeStruct((2, OUT1), f32),
        ],
    )(y0f, stats0, w1t, row(b1), row(g0), row(beta0))

    out = pl.pallas_call(
        _k3_body,
        grid=(nblk,),
        in_specs=[
            pl.BlockSpec((ROWBLK, OUT1), lambda i: (i, 0)),
            pl.BlockSpec((2, OUT1), lambda i: (0, 0)),
            pl.BlockSpec((1, OUT1), lambda i: (0, 0)),
            pl.BlockSpec((1, OUT1), lambda i: (0, 0)),
        ],
        out_specs=pl.BlockSpec((ROWBLK, OUT1), lambda i: (i, 0)),
        out_shape=jax.ShapeDtypeStruct((NROWS, OUT1), f32),
    )(y1, stats1, row(g1), row(beta1))

    return out.reshape(B, N, OUT1)


# speedup vs baseline: 17.6886x; 1.2962x over previous
import functools

import jax
import jax.numpy as jnp
from jax.experimental import pallas as pl
from jax.experimental.pallas import tpu as pltpu

B, N, M = 16, 1024, 256
C1, C2 = 256, 256
OUT0, OUT1 = 256, 256
NROWS = B * N
ROWBLK = 2048
EPS_BN = 1e-5
EPS_D = 1e-8


def _k1_body(x1_ref, x2t_ref, p1_ref, p2_ref, w0at_ref, w0bt_ref, b0_ref,
             y0_ref, stats_ref):
    b = pl.program_id(0)

    x1 = x1_ref[0]
    x2t = x2t_ref[0]
    x1sq = jnp.sum(x1 * x1, axis=1, keepdims=True)
    x2sq = jnp.sum(x2t * x2t, axis=0, keepdims=True)
    cross = jax.lax.dot_general(
        x1, x2t, (((1,), (0,)), ((), ())),
        preferred_element_type=jnp.float32,
        precision=jax.lax.Precision.DEFAULT)
    d2 = jnp.maximum(x1sq + x2sq - 2.0 * cross, 0.0)

    big = jnp.float32(3.4e38)
    s = jnp.zeros((N, M), jnp.float32)
    recip_sum = jnp.zeros((N, 1), jnp.float32)
    for _ in range(3):
        mval = jnp.min(d2, axis=1, keepdims=True)
        hit = d2 == mval
        r = 1.0 / (mval + EPS_D)
        recip_sum = recip_sum + r
        s = jnp.where(hit, r, s)
        d2 = jnp.where(hit, big, d2)
    s = s * (1.0 / recip_sum)

    z = jax.lax.dot_general(
        p2_ref[0], w0bt_ref[...], (((1,), (0,)), ((), ())),
        preferred_element_type=jnp.float32)
    y0 = (jax.lax.dot_general(p1_ref[0], w0at_ref[...],
                              (((1,), (0,)), ((), ())),
                              preferred_element_type=jnp.float32)
          + jax.lax.dot_general(s, z, (((1,), (0,)), ((), ())),
                                preferred_element_type=jnp.float32)
          + b0_ref[...])
    y0_ref[0] = y0

    @pl.when(b == 0)
    def _init():
        stats_ref[...] = jnp.zeros_like(stats_ref)

    part = jnp.concatenate(
        [jnp.sum(y0, axis=0, keepdims=True),
         jnp.sum(y0 * y0, axis=0, keepdims=True)], axis=0)
    stats_ref[...] += part


def _k2_body(y0_ref, stats0_ref, w1t_ref, b1_ref, g0_ref, beta0_ref,
             y1_ref, stats1_ref):
    i = pl.program_id(0)
    inv_n = jnp.float32(1.0 / NROWS)
    mean = stats0_ref[0:1, :] * inv_n
    var = stats0_ref[1:2, :] * inv_n - mean * mean
    scale = g0_ref[...] * jax.lax.rsqrt(var + EPS_BN)
    shift = beta0_ref[...] - mean * scale
    h = jnp.maximum(y0_ref[...] * scale + shift, 0.0)
    y1 = jax.lax.dot_general(h, w1t_ref[...], (((1,), (0,)), ((), ())),
                             preferred_element_type=jnp.float32) + b1_ref[...]
    y1_ref[...] = y1

    @pl.when(i == 0)
    def _init():
        stats1_ref[...] = jnp.zeros_like(stats1_ref)

    part = jnp.concatenate(
        [jnp.sum(y1, axis=0, keepdims=True),
         jnp.sum(y1 * y1, axis=0, keepdims=True)], axis=0)
    stats1_ref[...] += part


def _k3_body(y1_ref, stats1_ref, g1_ref, beta1_ref, out_ref):
    inv_n = jnp.float32(1.0 / NROWS)
    mean = stats1_ref[0:1, :] * inv_n
    var = stats1_ref[1:2, :] * inv_n - mean * mean
    scale = g1_ref[...] * jax.lax.rsqrt(var + EPS_BN)
    shift = beta1_ref[...] - mean * scale
    out_ref[...] = jnp.maximum(y1_ref[...] * scale + shift, 0.0)


@jax.jit
def kernel(xyz1, xyz2, points1, points2, W0, b0, g0, beta0, W1, b1, g1, beta1):
    f32 = jnp.float32
    x1p = jnp.pad(xyz1, ((0, 0), (0, 0), (0, 5)))
    x2t = jnp.pad(xyz2, ((0, 0), (0, 0), (0, 5))).transpose(0, 2, 1)
    w0t = W0.T
    w0at, w0bt = w0t[:C1], w0t[C1:]
    w1t = W1.T
    row = lambda v: v.reshape(1, -1)

    y0, stats0 = pl.pallas_call(
        _k1_body,
        grid=(B,),
        in_specs=[
            pl.BlockSpec((1, N, 8), lambda b: (b, 0, 0)),
            pl.BlockSpec((1, 8, M), lambda b: (b, 0, 0)),
            pl.BlockSpec((1, N, C1), lambda b: (b, 0, 0)),
            pl.BlockSpec((1, M, C2), lambda b: (b, 0, 0)),
            pl.BlockSpec((C1, OUT0), lambda b: (0, 0)),
            pl.BlockSpec((C2, OUT0), lambda b: (0, 0)),
            pl.BlockSpec((1, OUT0), lambda b: (0, 0)),
        ],
        out_specs=[
            pl.BlockSpec((1, N, OUT0), lambda b: (b, 0, 0)),
            pl.BlockSpec((2, OUT0), lambda b: (0, 0)),
        ],
        out_shape=[
            jax.ShapeDtypeStruct((B, N, OUT0), f32),
            jax.ShapeDtypeStruct((2, OUT0), f32),
        ],
    )(x1p, x2t, points1, points2, w0at, w0bt, row(b0))

    y0f = y0.reshape(NROWS, OUT0)
    nblk = NROWS // ROWBLK
    y1, stats1 = pl.pallas_call(
        _k2_body,
        grid=(nblk,),
        in_specs=[
            pl.BlockSpec((ROWBLK, OUT0), lambda i: (i, 0)),
            pl.BlockSpec((2, OUT0), lambda i: (0, 0)),
            pl.BlockSpec((OUT0, OUT1), lambda i: (0, 0)),
            pl.BlockSpec((1, OUT1), lambda i: (0, 0)),
            pl.BlockSpec((1, OUT0), lambda i: (0, 0)),
            pl.BlockSpec((1, OUT0), lambda i: (0, 0)),
        ],
        out_specs=[
            pl.BlockSpec((ROWBLK, OUT1), lambda i: (i, 0)),
            pl.BlockSpec((2, OUT1), lambda i: (0, 0)),
        ],
        out_shape=[
            jax.ShapeDtypeStruct((NROWS, OUT1), f32),
            jax.ShapeDtypeStruct((2, OUT1), f32),
        ],
    )(y0f, stats0, w1t, row(b1), row(g0), row(beta0))

    out = pl.pallas_call(
        _k3_body,
        grid=(nblk,),
        in_specs=[
            pl.BlockSpec((ROWBLK, OUT1), lambda i: (i, 0)),
            pl.BlockSpec((2, OUT1), lambda i: (0, 0)),
            pl.BlockSpec((1, OUT1), lambda i: (0, 0)),
            pl.BlockSpec((1, OUT1), lambda i: (0, 0)),
        ],
        out_specs=pl.BlockSpec((ROWBLK, OUT1), lambda i: (i, 0)),
        out_shape=jax.ShapeDtypeStruct((NROWS, OUT1), f32),
    )(y1, stats1, row(g1), row(beta1))

    return out.reshape(B, N, OUT1)
